# baseline (speedup 1.0000x reference)
import functools

import jax
import jax.numpy as jnp
from jax import lax
from jax.experimental import pallas as pl
from jax.experimental.pallas import tpu as pltpu
from jax.experimental.pallas import tpu_sc as plsc

B, L, V, D = 16384, 200, 23, 128
VP = 32
LANES = 16
NC, NS = 2, 16
NW = NC * NS
ROWS_PER_W = B // NW
CHUNK = 128
NCHUNK = ROWS_PER_W // CHUNK
GROUPS = CHUNK // LANES
LBLK = 20

_sc_mesh = plsc.VectorSubcoreMesh(
    core_axis_name="c", subcore_axis_name="s", num_cores=NC, num_subcores=NS)


@functools.partial(
    pl.kernel,
    out_type=jax.ShapeDtypeStruct((VP, B), jnp.float32),
    mesh=_sc_mesh,
    scratch_types=[
        pltpu.VMEM((L, ROWS_PER_W), jnp.int32),
        pltpu.VMEM((VP, CHUNK), jnp.float32),
        pltpu.SemaphoreType.DMA,
        pltpu.SemaphoreType.DMA,
    ],
    compiler_params=pltpu.CompilerParams(needs_layout_passes=False),
)
def _histogram_sc(xt_hbm, counts_hbm, x_v, c_v, sem0, sem1):
    wid = lax.axis_index("s") * NC + lax.axis_index("c")
    lane = lax.iota(jnp.int32, LANES)
    ones = jnp.ones((LANES,), jnp.float32)
    zeros = jnp.zeros((LANES,), jnp.float32)
    col_base = wid * ROWS_PER_W

    first = pltpu.async_copy(
        xt_hbm.at[:, pl.ds(col_base, CHUNK)],
        x_v.at[:, pl.ds(0, CHUNK)], sem0)
    rest = pltpu.async_copy(
        xt_hbm.at[:, pl.ds(col_base + CHUNK, ROWS_PER_W - CHUNK)],
        x_v.at[:, pl.ds(CHUNK, ROWS_PER_W - CHUNK)], sem1)
    first.wait()

    def chunk_compute(ci, carry):
        def zero_body(r, zcarry):
            for j in range(CHUNK // LANES):
                c_v[r, pl.ds(j * LANES, LANES)] = zeros
            return zcarry
        lax.fori_loop(0, VP, zero_body, 0)

        def group_body(g, gcarry):
            row = g * LANES + lane
            coff = ci * CHUNK + g * LANES
            xs = [x_v[j, pl.ds(coff, LANES)] for j in range(LBLK)]
            for lb in range(LBLK, L, LBLK):
                nxt = []
                for j in range(LBLK):
                    nxt.append(x_v[lb + j, pl.ds(coff, LANES)])
                    plsc.addupdate_scatter(c_v, [xs[j], row], ones)
                xs = nxt
            for j in range(LBLK):
                plsc.addupdate_scatter(c_v, [xs[j], row], ones)
            return gcarry

        lax.fori_loop(0, GROUPS, group_body, 0)
        pltpu.sync_copy(
            c_v, counts_hbm.at[:, pl.ds(col_base + ci * CHUNK, CHUNK)])
        return carry

    chunk_compute(0, 0)
    rest.wait()
    lax.fori_loop(1, NCHUNK, chunk_compute, 0)


_MM_BLK = 2048


def _tanh_poly(x):
    x = jnp.clip(x, -7.99881172180175781, 7.99881172180175781)
    a = x * x
    p = jnp.float32(-2.76076847742355e-16)
    p = p * a + jnp.float32(2.00018790482477e-13)
    p = p * a + jnp.float32(-8.60467152213735e-11)
    p = p * a + jnp.float32(5.12229709037114e-08)
    p = p * a + jnp.float32(1.48572235717979e-05)
    p = p * a + jnp.float32(6.37261928875436e-04)
    p = p * a + jnp.float32(4.89352455891786e-03)
    p = p * x
    q = jnp.float32(1.19825839466702e-06)
    q = q * a + jnp.float32(1.18534705686654e-04)
    q = q * a + jnp.float32(2.26843463243900e-03)
    q = q * a + jnp.float32(4.89352518554385e-03)
    return p / q


def _matmul_tanh_body(c_ref, t_ref, o_ref):
    acc = jax.lax.dot_general(
        c_ref[...], t_ref[...], (((0,), (0,)), ((), ())),
        precision=jax.lax.Precision.HIGHEST,
        preferred_element_type=jnp.float32)
    o_ref[...] = jnp.tanh(acc)


_matmul_tanh = pl.pallas_call(
    _matmul_tanh_body,
    grid=(B // _MM_BLK,),
    in_specs=[
        pl.BlockSpec((VP, _MM_BLK), lambda i: (0, i)),
        pl.BlockSpec((VP, D), lambda i: (0, 0)),
    ],
    out_specs=pl.BlockSpec((_MM_BLK, D), lambda i: (i, 0)),
    out_shape=jax.ShapeDtypeStruct((B, D), jnp.float32),
)


def kernel(x, table):
    xt = x.astype(jnp.int32).T
    counts = _histogram_sc(xt)
    t_pad = jnp.zeros((VP, D), jnp.float32).at[1:V].set(table[1:V])
    return _matmul_tanh(counts, t_pad)


# speedup vs baseline: 1.3001x; 1.3001x over previous
import functools

import jax
import jax.numpy as jnp
from jax import lax
from jax.experimental import pallas as pl
from jax.experimental.pallas import tpu as pltpu
from jax.experimental.pallas import tpu_sc as plsc

B, L, V, D = 16384, 200, 23, 128
VP = 32
LANES = 16
NC, NS = 2, 16
NW = NC * NS
ROWS_PER_W = B // NW
CHUNK = 128
NCHUNK = ROWS_PER_W // CHUNK
GROUPS = CHUNK // LANES
LBLK = 20

_sc_mesh = plsc.VectorSubcoreMesh(
    core_axis_name="c", subcore_axis_name="s", num_cores=NC, num_subcores=NS)


@functools.partial(
    pl.kernel,
    out_type=jax.ShapeDtypeStruct((VP, B), jnp.float32),
    mesh=_sc_mesh,
    scratch_types=[
        pltpu.VMEM((L, CHUNK), jnp.int32),
        pltpu.VMEM((VP, CHUNK), jnp.float32),
    ],
    compiler_params=pltpu.CompilerParams(needs_layout_passes=False),
)
def _histogram_sc(xt_hbm, counts_hbm, x_v, c_v):
    wid = lax.axis_index("s") * NC + lax.axis_index("c")
    lane = lax.iota(jnp.int32, LANES)
    ones = jnp.ones((LANES,), jnp.float32)
    zeros = jnp.zeros((LANES,), jnp.float32)

    def chunk_body(ci, carry):
        row_base = wid * ROWS_PER_W + ci * CHUNK
        pltpu.sync_copy(xt_hbm.at[:, pl.ds(row_base, CHUNK)], x_v)
        def zero_body(r, zcarry):
            for j in range(CHUNK // LANES):
                c_v[r, pl.ds(j * LANES, LANES)] = zeros
            return zcarry
        lax.fori_loop(0, VP, zero_body, 0)

        def group_body(g, gcarry):
            row = g * LANES + lane
            xs = [x_v[j, pl.ds(g * LANES, LANES)] for j in range(LBLK)]
            for lb in range(LBLK, L, LBLK):
                nxt = []
                for j in range(LBLK):
                    nxt.append(x_v[lb + j, pl.ds(g * LANES, LANES)])
                    plsc.addupdate_scatter(c_v, [xs[j], row], ones)
                xs = nxt
            for j in range(LBLK):
                plsc.addupdate_scatter(c_v, [xs[j], row], ones)
            return gcarry

        lax.fori_loop(0, GROUPS, group_body, 0)
        pltpu.sync_copy(c_v, counts_hbm.at[:, pl.ds(row_base, CHUNK)])
        return carry

    lax.fori_loop(0, NCHUNK, chunk_body, 0)


_MM_BLK = 2048


def _tanh_poly(x):
    x = jnp.clip(x, -7.99881172180175781, 7.99881172180175781)
    a = x * x
    p = jnp.float32(-2.76076847742355e-16)
    p = p * a + jnp.float32(2.00018790482477e-13)
    p = p * a + jnp.float32(-8.60467152213735e-11)
    p = p * a + jnp.float32(5.12229709037114e-08)
    p = p * a + jnp.float32(1.48572235717979e-05)
    p = p * a + jnp.float32(6.37261928875436e-04)
    p = p * a + jnp.float32(4.89352455891786e-03)
    p = p * x
    q = jnp.float32(1.19825839466702e-06)
    q = q * a + jnp.float32(1.18534705686654e-04)
    q = q * a + jnp.float32(2.26843463243900e-03)
    q = q * a + jnp.float32(4.89352518554385e-03)
    return p / q


def _matmul_tanh_body(c_ref, t_ref, o_ref):
    acc = jax.lax.dot_general(
        c_ref[...], t_ref[...], (((0,), (0,)), ((), ())),
        precision=jax.lax.Precision.HIGHEST,
        preferred_element_type=jnp.float32)
    o_ref[...] = jnp.tanh(acc)


_matmul_tanh = pl.pallas_call(
    _matmul_tanh_body,
    grid=(B // _MM_BLK,),
    in_specs=[
        pl.BlockSpec((VP, _MM_BLK), lambda i: (0, i)),
        pl.BlockSpec((VP, D), lambda i: (0, 0)),
    ],
    out_specs=pl.BlockSpec((_MM_BLK, D), lambda i: (i, 0)),
    out_shape=jax.ShapeDtypeStruct((B, D), jnp.float32),
)


def kernel(x, table):
    xt = x.astype(jnp.int32).T
    counts = _histogram_sc(xt)
    t_pad = jnp.zeros((VP, D), jnp.float32).at[1:V].set(table[1:V])
    return _matmul_tanh(counts, t_pad)


# speedup vs baseline: 1.3077x; 1.0059x over previous
import functools

import jax
import jax.numpy as jnp
from jax import lax
from jax.experimental import pallas as pl
from jax.experimental.pallas import tpu as pltpu
from jax.experimental.pallas import tpu_sc as plsc

B, L, V, D = 16384, 200, 23, 128
VP = 24
LANES = 16
NC, NS = 2, 16
NW = NC * NS
ROWS_PER_W = B // NW
CHUNK = 128
NCHUNK = ROWS_PER_W // CHUNK
GROUPS = CHUNK // LANES
LBLK = 25

_sc_mesh = plsc.VectorSubcoreMesh(
    core_axis_name="c", subcore_axis_name="s", num_cores=NC, num_subcores=NS)


@functools.partial(
    pl.kernel,
    out_type=jax.ShapeDtypeStruct((VP, B), jnp.float32),
    mesh=_sc_mesh,
    scratch_types=[
        pltpu.VMEM((L, CHUNK), jnp.int32),
        pltpu.VMEM((VP, CHUNK), jnp.float32),
    ],
    compiler_params=pltpu.CompilerParams(needs_layout_passes=False),
)
def _histogram_sc(xt_hbm, counts_hbm, x_v, c_v):
    wid = lax.axis_index("s") * NC + lax.axis_index("c")
    lane = lax.iota(jnp.int32, LANES)
    ones = jnp.ones((LANES,), jnp.float32)
    zeros = jnp.zeros((LANES,), jnp.float32)

    def chunk_body(ci, carry):
        row_base = wid * ROWS_PER_W + ci * CHUNK
        pltpu.sync_copy(xt_hbm.at[:, pl.ds(row_base, CHUNK)], x_v)
        for r in range(VP):
            for j in range(CHUNK // LANES):
                c_v[r, pl.ds(j * LANES, LANES)] = zeros

        def group_body(g, gcarry):
            row = g * LANES + lane
            xs = [x_v[j, pl.ds(g * LANES, LANES)] for j in range(LBLK)]
            for lb in range(LBLK, L, LBLK):
                nxt = []
                for j in range(LBLK):
                    nxt.append(x_v[lb + j, pl.ds(g * LANES, LANES)])
                    plsc.addupdate_scatter(c_v, [xs[j], row], ones)
                xs = nxt
            for j in range(LBLK):
                plsc.addupdate_scatter(c_v, [xs[j], row], ones)
            return gcarry

        lax.fori_loop(0, GROUPS, group_body, 0)
        pltpu.sync_copy(c_v, counts_hbm.at[:, pl.ds(row_base, CHUNK)])
        return carry

    lax.fori_loop(0, NCHUNK, chunk_body, 0)


_MM_BLK = 2048


def _tanh_poly(x):
    x = jnp.clip(x, -7.99881172180175781, 7.99881172180175781)
    a = x * x
    p = jnp.float32(-2.76076847742355e-16)
    p = p * a + jnp.float32(2.00018790482477e-13)
    p = p * a + jnp.float32(-8.60467152213735e-11)
    p = p * a + jnp.float32(5.12229709037114e-08)
    p = p * a + jnp.float32(1.48572235717979e-05)
    p = p * a + jnp.float32(6.37261928875436e-04)
    p = p * a + jnp.float32(4.89352455891786e-03)
    p = p * x
    q = jnp.float32(1.19825839466702e-06)
    q = q * a + jnp.float32(1.18534705686654e-04)
    q = q * a + jnp.float32(2.26843463243900e-03)
    q = q * a + jnp.float32(4.89352518554385e-03)
    return p / q


def _matmul_tanh_body(c_ref, t_ref, o_ref):
    acc = jax.lax.dot_general(
        c_ref[...], t_ref[...], (((0,), (0,)), ((), ())),
        precision=jax.lax.Precision.HIGHEST,
        preferred_element_type=jnp.float32)
    o_ref[...] = jnp.tanh(acc)


_matmul_tanh = pl.pallas_call(
    _matmul_tanh_body,
    grid=(B // _MM_BLK,),
    in_specs=[
        pl.BlockSpec((VP, _MM_BLK), lambda i: (0, i)),
        pl.BlockSpec((VP, D), lambda i: (0, 0)),
    ],
    out_specs=pl.BlockSpec((_MM_BLK, D), lambda i: (i, 0)),
    out_shape=jax.ShapeDtypeStruct((B, D), jnp.float32),
)


def kernel(x, table):
    xt = x.astype(jnp.int32).T
    counts = _histogram_sc(xt)
    t_pad = jnp.zeros((VP, D), jnp.float32).at[1:V].set(table[1:V])
    return _matmul_tanh(counts, t_pad)


# speedup vs baseline: 1.3098x; 1.0016x over previous
import functools

import jax
import jax.numpy as jnp
from jax import lax
from jax.experimental import pallas as pl
from jax.experimental.pallas import tpu as pltpu
from jax.experimental.pallas import tpu_sc as plsc

B, L, V, D = 16384, 200, 23, 128
VP = 24
LANES = 16
NC, NS = 2, 16
NW = NC * NS
ROWS_PER_W = B // NW
CHUNK = 128
NCHUNK = ROWS_PER_W // CHUNK
GROUPS = CHUNK // LANES
LBLK = 25

_sc_mesh = plsc.VectorSubcoreMesh(
    core_axis_name="c", subcore_axis_name="s", num_cores=NC, num_subcores=NS)


@functools.partial(
    pl.kernel,
    out_type=jax.ShapeDtypeStruct((VP, B), jnp.float32),
    mesh=_sc_mesh,
    scratch_types=[
        pltpu.VMEM((L, CHUNK), jnp.int32),
        pltpu.VMEM((VP, CHUNK), jnp.float32),
    ],
    compiler_params=pltpu.CompilerParams(needs_layout_passes=False),
)
def _histogram_sc(xt_hbm, counts_hbm, x_v, c_v):
    wid = lax.axis_index("s") * NC + lax.axis_index("c")
    lane = lax.iota(jnp.int32, LANES)
    ones = jnp.ones((LANES,), jnp.float32)
    zeros = jnp.zeros((LANES,), jnp.float32)

    def chunk_body(ci, carry):
        row_base = wid * ROWS_PER_W + ci * CHUNK
        pltpu.sync_copy(xt_hbm.at[:, pl.ds(row_base, CHUNK)], x_v)
        for r in range(VP):
            for j in range(CHUNK // LANES):
                c_v[r, pl.ds(j * LANES, LANES)] = zeros

        def group_body(g, gcarry):
            row = g * LANES + lane
            xs = [x_v[j, pl.ds(g * LANES, LANES)] for j in range(LBLK)]
            for lb in range(LBLK, L, LBLK):
                nxt = []
                for j in range(LBLK):
                    nxt.append(x_v[lb + j, pl.ds(g * LANES, LANES)])
                    plsc.addupdate_scatter(c_v, [xs[j], row], ones)
                xs = nxt
            for j in range(LBLK):
                plsc.addupdate_scatter(c_v, [xs[j], row], ones)
            return gcarry

        lax.fori_loop(0, GROUPS, group_body, 0)
        pltpu.sync_copy(c_v, counts_hbm.at[:, pl.ds(row_base, CHUNK)])
        return carry

    lax.fori_loop(0, NCHUNK, chunk_body, 0)


_MM_BLK = 2048


def _matmul_tanh_body(c_ref, t_ref, o_ref):
    acc = jax.lax.dot_general(
        c_ref[...], t_ref[...], (((0,), (0,)), ((), ())),
        precision=jax.lax.Precision.HIGHEST,
        preferred_element_type=jnp.float32)
    o_ref[...] = jnp.tanh(acc)


_matmul_tanh = pl.pallas_call(
    _matmul_tanh_body,
    grid=(B // _MM_BLK,),
    in_specs=[
        pl.BlockSpec((VP, _MM_BLK), lambda i: (0, i)),
        pl.BlockSpec((VP, D), lambda i: (0, 0)),
    ],
    out_specs=pl.BlockSpec((_MM_BLK, D), lambda i: (i, 0)),
    out_shape=jax.ShapeDtypeStruct((B, D), jnp.float32),
)


def kernel(x, table):
    xt = x.astype(jnp.int32).T
    counts = _histogram_sc(xt)
    t_pad = jnp.zeros((VP, D), jnp.float32).at[1:V].set(table[1:V])
    return _matmul_tanh(counts, t_pad)
